# baseline (speedup 1.0000x reference)
import functools

import jax
import jax.numpy as jnp
from jax import lax
from jax.experimental import pallas as pl
from jax.experimental.pallas import tpu as pltpu
from jax.experimental.pallas import tpu_sc as plsc

_B, _C, _K = 4096, 1000, 10
_EPS = 1e-7

_NC, _NS, _L = 2, 16, 16
_NW = _NC * _NS
_SPW = _B // _NW

_BLOCK_B = 1024


_CCHUNK = 288
_CHUNKS = (64, 128, 256, 288, 264)


def _sc_gather_body(xt_hbm, labt_hbm, out_hbm, x0_v, x1_v, lab_v, g_v,
                    sem0, sem1):
    wid = lax.axis_index("s") * _NC + lax.axis_index("c")
    i0 = wid * _SPW

    bufs = (x0_v, x1_v)
    sems = (sem0, sem1)
    starts = [sum(_CHUNKS[:p]) for p in range(len(_CHUNKS))]
    copies = [None] * len(_CHUNKS)
    copies[0] = pltpu.async_copy(
        xt_hbm.at[pl.ds(0, _CHUNKS[0]), pl.ds(i0, _SPW)],
        x0_v.at[pl.ds(0, _CHUNKS[0])], sems[0])
    pltpu.sync_copy(labt_hbm.at[:, pl.ds(i0, _SPW)], lab_v)

    iota = lax.iota(jnp.int32, _L)

    for p, csz in enumerate(_CHUNKS):
        if p + 1 < len(_CHUNKS):
            nxt = (p + 1) % 2
            copies[p + 1] = pltpu.async_copy(
                xt_hbm.at[pl.ds(starts[p + 1], _CHUNKS[p + 1]),
                          pl.ds(i0, _SPW)],
                bufs[nxt].at[pl.ds(0, _CHUNKS[p + 1])], sems[nxt])
        copies[p].wait()
        xs = bufs[p % 2]
        c0 = starts[p]

        def _chunk(c, carry, *, xs=xs, c0=c0, csz=csz, p=p):
            ivec = c * _L + iota
            for k in range(_K):
                lab = lab_v[k, pl.ds(c * _L, _L)] - c0
                idx = jnp.minimum(jnp.maximum(lab, 0), csz - 1)
                g = plsc.load_gather(xs, [idx, ivec])
                hit = (lab >= 0) & (lab < csz)
                if p == 0:
                    g_v[k, pl.ds(c * _L, _L)] = jnp.where(hit, g, 0.0)
                else:
                    prev = g_v[k, pl.ds(c * _L, _L)]
                    g_v[k, pl.ds(c * _L, _L)] = jnp.where(hit, g, prev)
            return carry

        lax.fori_loop(0, _SPW // _L, _chunk, 0)

    pltpu.sync_copy(g_v, out_hbm.at[:, pl.ds(i0, _SPW)])


@functools.cache
def _sc_gather():
    return pl.kernel(
        _sc_gather_body,
        out_type=jax.ShapeDtypeStruct((_K, _B), jnp.float32),
        mesh=plsc.VectorSubcoreMesh(core_axis_name="c", subcore_axis_name="s"),
        compiler_params=pltpu.CompilerParams(needs_layout_passes=False),
        scratch_types=[
            pltpu.VMEM((_CCHUNK, _SPW), jnp.float32),
            pltpu.VMEM((_CCHUNK, _SPW), jnp.float32),
            pltpu.VMEM((_K, _SPW), jnp.int32),
            pltpu.VMEM((_K, _SPW), jnp.float32),
            pltpu.SemaphoreType.DMA,
            pltpu.SemaphoreType.DMA,
        ],
    )


def _lse_kernel(xt_ref, lse_ref):
    x = xt_ref[:]
    m = jnp.max(x, axis=0)
    z = jnp.sum(jnp.exp(x - m[None, :]), axis=0)
    lse_ref[0, :] = m + jnp.log(z)


def _combine_kernel(g_ref, lab_ref, lse_ref, out_ref):
    probs = jnp.exp(g_ref[:] - lse_ref[:])
    lab = lab_ref[:]
    s = probs[0:1, :]
    for k in range(1, _K):
        dup = lab[k : k + 1, :] == lab[0:1, :]
        for j in range(1, k):
            dup = dup | (lab[k : k + 1, :] == lab[j : j + 1, :])
        s = s + jnp.where(dup, 0.0, probs[k : k + 1, :])
    loss = -jnp.log(1.0 - s + _EPS)
    out_ref[:, :] = (jnp.sum(loss) * (1.0 / _B)).reshape(1, 1)


@jax.jit
def kernel(outputs, complementary_labels):
    xt = pltpu.with_memory_space_constraint(outputs.T, pltpu.MemorySpace.HBM)
    labt = complementary_labels.T

    lse = pl.pallas_call(
        _lse_kernel,
        grid=(_B // _BLOCK_B,),
        in_specs=[pl.BlockSpec((_C, _BLOCK_B), lambda i: (0, i))],
        out_specs=pl.BlockSpec((1, _BLOCK_B), lambda i: (0, i)),
        out_shape=jax.ShapeDtypeStruct((1, _B), jnp.float32),
    )(xt)

    g = _sc_gather()(xt, labt)

    out = pl.pallas_call(
        _combine_kernel,
        in_specs=[
            pl.BlockSpec((_K, _B), lambda: (0, 0)),
            pl.BlockSpec((_K, _B), lambda: (0, 0)),
            pl.BlockSpec((1, _B), lambda: (0, 0)),
        ],
        out_specs=pl.BlockSpec((1, 1), lambda: (0, 0)),
        out_shape=jax.ShapeDtypeStruct((1, 1), jnp.float32),
    )(g, labt, lse)
    return out[0, 0]


# speedup vs baseline: 1.0047x; 1.0047x over previous
import functools

import jax
import jax.numpy as jnp
from jax import lax
from jax.experimental import pallas as pl
from jax.experimental.pallas import tpu as pltpu
from jax.experimental.pallas import tpu_sc as plsc

_B, _C, _K = 4096, 1000, 10
_EPS = 1e-7

_NC, _NS, _L = 2, 16, 16
_NW = _NC * _NS
_SPW = _B // _NW

_BLOCK_B = 1024


_CCHUNK = 256
_CHUNKS = (128, 256, 256, 256, 104)


def _sc_gather_body(xt_hbm, labt_hbm, out_hbm, x0_v, x1_v, lab_v, g_v,
                    sem0, sem1):
    wid = lax.axis_index("s") * _NC + lax.axis_index("c")
    i0 = wid * _SPW

    bufs = (x0_v, x1_v)
    sems = (sem0, sem1)
    starts = [sum(_CHUNKS[:p]) for p in range(len(_CHUNKS))]
    copies = [None] * len(_CHUNKS)
    copies[0] = pltpu.async_copy(
        xt_hbm.at[pl.ds(0, _CHUNKS[0]), pl.ds(i0, _SPW)],
        x0_v.at[pl.ds(0, _CHUNKS[0])], sems[0])
    pltpu.sync_copy(labt_hbm.at[:, pl.ds(i0, _SPW)], lab_v)

    iota = lax.iota(jnp.int32, _L)

    for p, csz in enumerate(_CHUNKS):
        if p + 1 < len(_CHUNKS):
            nxt = (p + 1) % 2
            copies[p + 1] = pltpu.async_copy(
                xt_hbm.at[pl.ds(starts[p + 1], _CHUNKS[p + 1]),
                          pl.ds(i0, _SPW)],
                bufs[nxt].at[pl.ds(0, _CHUNKS[p + 1])], sems[nxt])
        copies[p].wait()
        xs = bufs[p % 2]
        c0 = starts[p]

        def _chunk(c, carry, *, xs=xs, c0=c0, csz=csz, p=p):
            ivec = c * _L + iota
            for k in range(_K):
                lab = lab_v[k, pl.ds(c * _L, _L)] - c0
                idx = jnp.minimum(jnp.maximum(lab, 0), csz - 1)
                g = plsc.load_gather(xs, [idx, ivec])
                hit = (lab >= 0) & (lab < csz)
                if p == 0:
                    g_v[k, pl.ds(c * _L, _L)] = jnp.where(hit, g, 0.0)
                else:
                    prev = g_v[k, pl.ds(c * _L, _L)]
                    g_v[k, pl.ds(c * _L, _L)] = jnp.where(hit, g, prev)
            return carry

        lax.fori_loop(0, _SPW // _L, _chunk, 0)

    pltpu.sync_copy(g_v, out_hbm.at[:, pl.ds(i0, _SPW)])


@functools.cache
def _sc_gather():
    return pl.kernel(
        _sc_gather_body,
        out_type=jax.ShapeDtypeStruct((_K, _B), jnp.float32),
        mesh=plsc.VectorSubcoreMesh(core_axis_name="c", subcore_axis_name="s"),
        compiler_params=pltpu.CompilerParams(needs_layout_passes=False),
        scratch_types=[
            pltpu.VMEM((_CCHUNK, _SPW), jnp.float32),
            pltpu.VMEM((_CCHUNK, _SPW), jnp.float32),
            pltpu.VMEM((_K, _SPW), jnp.int32),
            pltpu.VMEM((_K, _SPW), jnp.float32),
            pltpu.SemaphoreType.DMA,
            pltpu.SemaphoreType.DMA,
        ],
    )


def _lse_kernel(xt_ref, lse_ref):
    x = xt_ref[:]
    m = jnp.max(x, axis=0)
    z = jnp.sum(jnp.exp(x - m[None, :]), axis=0)
    lse_ref[0, :] = m + jnp.log(z)


def _combine_kernel(g_ref, lab_ref, lse_ref, out_ref):
    probs = jnp.exp(g_ref[:] - lse_ref[:])
    lab = lab_ref[:]
    s = probs[0:1, :]
    for k in range(1, _K):
        dup = lab[k : k + 1, :] == lab[0:1, :]
        for j in range(1, k):
            dup = dup | (lab[k : k + 1, :] == lab[j : j + 1, :])
        s = s + jnp.where(dup, 0.0, probs[k : k + 1, :])
    loss = -jnp.log(1.0 - s + _EPS)
    out_ref[:, :] = (jnp.sum(loss) * (1.0 / _B)).reshape(1, 1)


@jax.jit
def kernel(outputs, complementary_labels):
    xt = pltpu.with_memory_space_constraint(outputs.T, pltpu.MemorySpace.HBM)
    labt = complementary_labels.T

    lse = pl.pallas_call(
        _lse_kernel,
        grid=(_B // _BLOCK_B,),
        in_specs=[pl.BlockSpec((_C, _BLOCK_B), lambda i: (0, i))],
        out_specs=pl.BlockSpec((1, _BLOCK_B), lambda i: (0, i)),
        out_shape=jax.ShapeDtypeStruct((1, _B), jnp.float32),
    )(xt)

    g = _sc_gather()(xt, labt)

    out = pl.pallas_call(
        _combine_kernel,
        in_specs=[
            pl.BlockSpec((_K, _B), lambda: (0, 0)),
            pl.BlockSpec((_K, _B), lambda: (0, 0)),
            pl.BlockSpec((1, _B), lambda: (0, 0)),
        ],
        out_specs=pl.BlockSpec((1, 1), lambda: (0, 0)),
        out_shape=jax.ShapeDtypeStruct((1, 1), jnp.float32),
    )(g, labt, lse)
    return out[0, 0]


# speedup vs baseline: 1.0058x; 1.0011x over previous
import functools

import jax
import jax.numpy as jnp
from jax import lax
from jax.experimental import pallas as pl
from jax.experimental.pallas import tpu as pltpu
from jax.experimental.pallas import tpu_sc as plsc

_B, _C, _K = 4096, 1000, 10
_EPS = 1e-7

_NC, _NS, _L = 2, 16, 16
_NW = _NC * _NS
_SPW = _B // _NW

_BLOCK_B = 1024


_CCHUNK = 256
_CHUNKS = (128, 256, 256, 256, 104)


def _sc_gather_body(xt_hbm, labt_hbm, out_hbm, x0_v, x1_v, lab_v, g_v,
                    sem0, sem1):
    wid = lax.axis_index("c") * _NS + lax.axis_index("s")
    i0 = wid * _SPW

    bufs = (x0_v, x1_v)
    sems = (sem0, sem1)
    starts = [sum(_CHUNKS[:p]) for p in range(len(_CHUNKS))]
    copies = [None] * len(_CHUNKS)
    copies[0] = pltpu.async_copy(
        xt_hbm.at[pl.ds(0, _CHUNKS[0]), pl.ds(i0, _SPW)],
        x0_v.at[pl.ds(0, _CHUNKS[0])], sems[0])
    pltpu.sync_copy(labt_hbm.at[:, pl.ds(i0, _SPW)], lab_v)

    iota = lax.iota(jnp.int32, _L)

    for p, csz in enumerate(_CHUNKS):
        if p + 1 < len(_CHUNKS):
            nxt = (p + 1) % 2
            copies[p + 1] = pltpu.async_copy(
                xt_hbm.at[pl.ds(starts[p + 1], _CHUNKS[p + 1]),
                          pl.ds(i0, _SPW)],
                bufs[nxt].at[pl.ds(0, _CHUNKS[p + 1])], sems[nxt])
        copies[p].wait()
        xs = bufs[p % 2]
        c0 = starts[p]

        def _chunk(c, carry, *, xs=xs, c0=c0, csz=csz, p=p):
            ivec = c * _L + iota
            for k in range(_K):
                lab = lab_v[k, pl.ds(c * _L, _L)] - c0
                idx = jnp.minimum(jnp.maximum(lab, 0), csz - 1)
                g = plsc.load_gather(xs, [idx, ivec])
                hit = (lab >= 0) & (lab < csz)
                if p == 0:
                    g_v[k, pl.ds(c * _L, _L)] = jnp.where(hit, g, 0.0)
                else:
                    prev = g_v[k, pl.ds(c * _L, _L)]
                    g_v[k, pl.ds(c * _L, _L)] = jnp.where(hit, g, prev)
            return carry

        lax.fori_loop(0, _SPW // _L, _chunk, 0)

    pltpu.sync_copy(g_v, out_hbm.at[:, pl.ds(i0, _SPW)])


@functools.cache
def _sc_gather():
    return pl.kernel(
        _sc_gather_body,
        out_type=jax.ShapeDtypeStruct((_K, _B), jnp.float32),
        mesh=plsc.VectorSubcoreMesh(core_axis_name="c", subcore_axis_name="s"),
        compiler_params=pltpu.CompilerParams(needs_layout_passes=False),
        scratch_types=[
            pltpu.VMEM((_CCHUNK, _SPW), jnp.float32),
            pltpu.VMEM((_CCHUNK, _SPW), jnp.float32),
            pltpu.VMEM((_K, _SPW), jnp.int32),
            pltpu.VMEM((_K, _SPW), jnp.float32),
            pltpu.SemaphoreType.DMA,
            pltpu.SemaphoreType.DMA,
        ],
    )


def _lse_kernel(xt_ref, lse_ref):
    x = xt_ref[:]
    m = jnp.max(x, axis=0)
    z = jnp.sum(jnp.exp(x - m[None, :]), axis=0)
    lse_ref[0, :] = m + jnp.log(z)


def _combine_kernel(g_ref, lab_ref, lse_ref, out_ref):
    probs = jnp.exp(g_ref[:] - lse_ref[:])
    lab = lab_ref[:]
    s = probs[0:1, :]
    for k in range(1, _K):
        dup = lab[k : k + 1, :] == lab[0:1, :]
        for j in range(1, k):
            dup = dup | (lab[k : k + 1, :] == lab[j : j + 1, :])
        s = s + jnp.where(dup, 0.0, probs[k : k + 1, :])
    loss = -jnp.log(1.0 - s + _EPS)
    out_ref[:, :] = (jnp.sum(loss) * (1.0 / _B)).reshape(1, 1)


@jax.jit
def kernel(outputs, complementary_labels):
    xt = pltpu.with_memory_space_constraint(outputs.T, pltpu.MemorySpace.HBM)
    labt = complementary_labels.T

    lse = pl.pallas_call(
        _lse_kernel,
        grid=(_B // _BLOCK_B,),
        in_specs=[pl.BlockSpec((_C, _BLOCK_B), lambda i: (0, i))],
        out_specs=pl.BlockSpec((1, _BLOCK_B), lambda i: (0, i)),
        out_shape=jax.ShapeDtypeStruct((1, _B), jnp.float32),
    )(xt)

    g = _sc_gather()(xt, labt)

    out = pl.pallas_call(
        _combine_kernel,
        in_specs=[
            pl.BlockSpec((_K, _B), lambda: (0, 0)),
            pl.BlockSpec((_K, _B), lambda: (0, 0)),
            pl.BlockSpec((1, _B), lambda: (0, 0)),
        ],
        out_specs=pl.BlockSpec((1, 1), lambda: (0, 0)),
        out_shape=jax.ShapeDtypeStruct((1, 1), jnp.float32),
    )(g, labt, lse)
    return out[0, 0]
